# baseline (speedup 1.0000x reference)
import functools

import jax
import jax.numpy as jnp
from jax import lax
from jax.experimental import pallas as pl
from jax.experimental.pallas import tpu as pltpu
from jax.experimental.pallas import tpu_sc as plsc

_LANES = 128
_K = 5
_CHUNK = _K * _LANES


@functools.lru_cache(maxsize=None)
def _make_gather(total_rows: int, hidden: int):
    info = plsc.get_sparse_core_info()
    nc, ns = info.num_cores, info.num_subcores
    nw = nc * ns
    rows_per_w = total_rows // nw
    n_chunks = rows_per_w // _CHUNK
    assert rows_per_w % _CHUNK == 0

    mesh = plsc.VectorSubcoreMesh(core_axis_name="c", subcore_axis_name="s")

    @functools.partial(
        pl.kernel,
        mesh=mesh,
        out_type=jax.ShapeDtypeStruct((total_rows, hidden), jnp.float32),
        scratch_types=[
            pltpu.VMEM((_CHUNK,), jnp.int32),
            pltpu.VMEM((_CHUNK, hidden), jnp.float32),
            pltpu.SemaphoreType.DMA,
        ],
        compiler_params=pltpu.CompilerParams(use_tc_tiling_on_sc=False),
    )
    def k(table_hbm, idx_hbm, out_hbm, idx_v, rows_v, gsem):
        wid = lax.axis_index("s") * nc + lax.axis_index("c")
        out0 = wid * rows_per_w

        def chunk_body(g, carry):
            pltpu.sync_copy(idx_hbm.at[pl.ds(out0 + g * _CHUNK, _CHUNK)], idx_v)
            cps = [
                pltpu.async_copy(
                    table_hbm.at[idx_v.at[pl.ds(j * _LANES, _LANES)]],
                    rows_v.at[pl.ds(j * _LANES, _LANES)],
                    gsem,
                )
                for j in range(_K)
            ]
            for cp in cps:
                cp.wait()
            pltpu.sync_copy(rows_v, out_hbm.at[pl.ds(out0 + g * _CHUNK, _CHUNK)])
            return carry

        lax.fori_loop(0, n_chunks, chunk_body, 0)

    return k


def kernel(inputs, embed_table):
    b, s = inputs.shape
    v, d = embed_table.shape
    total = b * s
    idx_flat = inputs.reshape(total).astype(jnp.int32)
    out = _make_gather(total, d)(embed_table, idx_flat)
    return out.reshape(b, s, d)


# speedup vs baseline: 1.0365x; 1.0365x over previous
import functools

import jax
import jax.numpy as jnp
from jax import lax
from jax.experimental import pallas as pl
from jax.experimental.pallas import tpu as pltpu
from jax.experimental.pallas import tpu_sc as plsc

_LANES = 128
_K = 5
_CHUNK = _K * _LANES


@functools.lru_cache(maxsize=None)
def _make_gather(total_rows: int, hidden: int):
    info = plsc.get_sparse_core_info()
    nc, ns = info.num_cores, info.num_subcores
    nw = nc * ns
    rows_per_w = total_rows // nw
    n_chunks = rows_per_w // _CHUNK
    n_pairs = n_chunks // 2
    assert rows_per_w % _CHUNK == 0 and n_chunks % 2 == 0

    mesh = plsc.VectorSubcoreMesh(core_axis_name="c", subcore_axis_name="s")

    @functools.partial(
        pl.kernel,
        mesh=mesh,
        out_type=jax.ShapeDtypeStruct((total_rows, hidden), jnp.float32),
        scratch_types=[
            pltpu.VMEM((rows_per_w,), jnp.int32),
            pltpu.VMEM((_CHUNK, hidden), jnp.float32),
            pltpu.VMEM((_CHUNK, hidden), jnp.float32),
            pltpu.SemaphoreType.DMA,
            pltpu.SemaphoreType.DMA,
        ],
        compiler_params=pltpu.CompilerParams(use_tc_tiling_on_sc=False),
    )
    def k(table_hbm, idx_hbm, out_hbm, idx_v, rows0, rows1, sem0, sem1):
        wid = lax.axis_index("s") * nc + lax.axis_index("c")
        out0 = wid * rows_per_w

        pltpu.sync_copy(idx_hbm.at[pl.ds(out0, rows_per_w)], idx_v)

        def fire(g, rows_v, sem):
            return [
                pltpu.async_copy(
                    table_hbm.at[idx_v.at[pl.ds(g * _CHUNK + j * _LANES, _LANES)]],
                    rows_v.at[pl.ds(j * _LANES, _LANES)],
                    sem,
                )
                for j in range(_K)
            ]

        def drain_wait(sem):
            for j in range(_K):
                pltpu.make_async_copy(
                    table_hbm.at[idx_v.at[pl.ds(j * _LANES, _LANES)]],
                    rows0.at[pl.ds(j * _LANES, _LANES)],
                    sem,
                ).wait()

        fire(0, rows0, sem0)

        def body(p, carry):
            g = 2 * p
            fire(g + 1, rows1, sem1)
            drain_wait(sem0)
            pltpu.sync_copy(rows0, out_hbm.at[pl.ds(out0 + g * _CHUNK, _CHUNK)])
            fire(g + 2, rows0, sem0)
            drain_wait(sem1)
            pltpu.sync_copy(rows1, out_hbm.at[pl.ds(out0 + (g + 1) * _CHUNK, _CHUNK)])
            return carry

        lax.fori_loop(0, n_pairs - 1, body, 0)

        g_last = n_chunks - 2
        fire(g_last + 1, rows1, sem1)
        drain_wait(sem0)
        pltpu.sync_copy(rows0, out_hbm.at[pl.ds(out0 + g_last * _CHUNK, _CHUNK)])
        drain_wait(sem1)
        pltpu.sync_copy(rows1, out_hbm.at[pl.ds(out0 + (g_last + 1) * _CHUNK, _CHUNK)])

    return k


def kernel(inputs, embed_table):
    b, s = inputs.shape
    v, d = embed_table.shape
    total = b * s
    idx_flat = inputs.reshape(total).astype(jnp.int32)
    out = _make_gather(total, d)(embed_table, idx_flat)
    return out.reshape(b, s, d)


# speedup vs baseline: 1.2627x; 1.2182x over previous
import functools

import jax
import jax.numpy as jnp
from jax import lax
from jax.experimental import pallas as pl
from jax.experimental.pallas import tpu as pltpu
from jax.experimental.pallas import tpu_sc as plsc

_LANES = 128
_K = 2
_CHUNK = _K * _LANES
_PADW = 128


@functools.lru_cache(maxsize=None)
def _make_gather(total_rows: int, hidden: int):
    info = plsc.get_sparse_core_info()
    nc, ns = info.num_cores, info.num_subcores
    nw = nc * ns
    rows_per_w = total_rows // nw
    n_chunks = rows_per_w // _CHUNK
    n_pairs = n_chunks // 2
    assert rows_per_w % _CHUNK == 0 and n_chunks % 2 == 0

    mesh = plsc.VectorSubcoreMesh(core_axis_name="c", subcore_axis_name="s")

    @functools.partial(
        pl.kernel,
        mesh=mesh,
        out_type=jax.ShapeDtypeStruct((total_rows, _PADW), jnp.float32),
        scratch_types=[
            pltpu.VMEM((rows_per_w,), jnp.int32),
            pltpu.VMEM((_CHUNK, _PADW), jnp.float32),
            pltpu.VMEM((_CHUNK, _PADW), jnp.float32),
            pltpu.SemaphoreType.DMA,
            pltpu.SemaphoreType.DMA,
        ],
    )
    def k(table_hbm, idx_hbm, out_hbm, idx_v, rows0, rows1, sem0, sem1):
        wid = lax.axis_index("s") * nc + lax.axis_index("c")
        out0 = wid * rows_per_w

        pltpu.sync_copy(idx_hbm.at[pl.ds(out0, rows_per_w)], idx_v)

        def fire(g, rows_v, sem):
            for j in range(_K):
                pltpu.async_copy(
                    table_hbm.at[idx_v.at[pl.ds(g * _CHUNK + j * _LANES, _LANES)]],
                    rows_v.at[pl.ds(j * _LANES, _LANES)],
                    sem,
                )

        def drain_wait(sem):
            for j in range(_K):
                pltpu.make_async_copy(
                    table_hbm.at[idx_v.at[pl.ds(j * _LANES, _LANES)]],
                    rows0.at[pl.ds(j * _LANES, _LANES)],
                    sem,
                ).wait()

        def store(g, rows_v):
            pltpu.sync_copy(
                rows_v,
                out_hbm.at[pl.ds(out0 + g * _CHUNK, _CHUNK)],
            )

        fire(0, rows0, sem0)

        def body(p, carry):
            g = 2 * p
            fire(g + 1, rows1, sem1)
            drain_wait(sem0)
            store(g, rows0)
            fire(g + 2, rows0, sem0)
            drain_wait(sem1)
            store(g + 1, rows1)
            return carry

        lax.fori_loop(0, n_pairs - 1, body, 0)

        g_last = n_chunks - 2
        fire(g_last + 1, rows1, sem1)
        drain_wait(sem0)
        store(g_last, rows0)
        drain_wait(sem1)
        store(g_last + 1, rows1)

    return k


def kernel(inputs, embed_table):
    b, s = inputs.shape
    v, d = embed_table.shape
    total = b * s
    idx_flat = inputs.reshape(total).astype(jnp.int32)
    tab_p = jnp.pad(embed_table, ((0, 0), (0, _PADW - d)))
    out = _make_gather(total, d)(tab_p, idx_flat)
    return out[:, :d].reshape(b, s, d)
